# baseline (speedup 1.0000x reference)
import dataclasses
import functools

import jax
import jax.numpy as jnp
from jax import lax
from jax.experimental import pallas as pl
from jax.experimental.pallas import tpu as pltpu
from jax.experimental.pallas import tpu_sc as plsc

N = 10000
E = 320000
G = 256
H = 128
F32 = jnp.float32

_NT = 5
_NROW = N // _NT
_EROW = 2560

_CHUNK = 128
_CPT = 78
_RPT = 624
BF16 = jnp.bfloat16


def _mm_bias_kernel(x_ref, w_ref, b_ref, o_ref):
    o_ref[...] = jnp.dot(x_ref[...], w_ref[...],
                         preferred_element_type=F32) + b_ref[...]


def _node_embed(x, w, b):
    return pl.pallas_call(
        _mm_bias_kernel,
        grid=(_NT,),
        in_specs=[
            pl.BlockSpec((_NROW, H), lambda i: (i, 0)),
            pl.BlockSpec((H, H), lambda i: (0, 0)),
            pl.BlockSpec((1, H), lambda i: (0, 0)),
        ],
        out_specs=pl.BlockSpec((_NROW, H), lambda i: (i, 0)),
        out_shape=jax.ShapeDtypeStruct((N, H), F32),
    )(x, w, b.reshape(1, H))


def _edge_mlp_kernel(a_ref, w1_ref, b1_ref, w2_ref, b2_ref, o_ref):
    t = jnp.maximum(jnp.dot(a_ref[...], w1_ref[...],
                            preferred_element_type=F32) + b1_ref[...], 0.0)
    o_ref[...] = (jnp.dot(t, w2_ref[...],
                          preferred_element_type=F32)
                  + b2_ref[...]).astype(BF16)


def _edge_mlp(a, w1, b1, w2, b2):
    d = a.shape[1]
    ne = a.shape[0]
    return pl.pallas_call(
        _edge_mlp_kernel,
        grid=(ne // _EROW,),
        in_specs=[
            pl.BlockSpec((_EROW, d), lambda i: (i, 0)),
            pl.BlockSpec((d, H), lambda i: (0, 0)),
            pl.BlockSpec((1, H), lambda i: (0, 0)),
            pl.BlockSpec((H, H), lambda i: (0, 0)),
            pl.BlockSpec((1, H), lambda i: (0, 0)),
        ],
        out_specs=pl.BlockSpec((_EROW, H), lambda i: (i, 0)),
        out_shape=jax.ShapeDtypeStruct((ne, H), BF16),
    )(a, w1, b1.reshape(1, H), w2, b2.reshape(1, H))


def _node_layer_kernel(h_ref, p0_ref, p1_ref, w1_ref, b1_ref, w2_ref, b2_ref,
                       t_ref, stats_ref, ssum, ssq):
    i = pl.program_id(0)

    @pl.when(i == 0)
    def _():
        ssum[...] = jnp.zeros_like(ssum)
        ssq[...] = jnp.zeros_like(ssq)

    z = h_ref[...] + p0_ref[...] + p1_ref[...]
    t = jnp.maximum(jnp.dot(z, w1_ref[...],
                            preferred_element_type=F32) + b1_ref[...], 0.0)
    t = jnp.dot(t, w2_ref[...], preferred_element_type=F32) + b2_ref[...]
    t_ref[...] = t
    ssum[...] += jnp.sum(t, axis=0, keepdims=True)
    ssq[...] += jnp.sum(t * t, axis=0, keepdims=True)

    @pl.when(i == _NT - 1)
    def _():
        stats_ref[0:1, :] = ssum[...]
        stats_ref[1:2, :] = ssq[...]


def _node_layer(h, p0, p1, w1, b1, w2, b2):
    return pl.pallas_call(
        _node_layer_kernel,
        grid=(_NT,),
        in_specs=[
            pl.BlockSpec((_NROW, H), lambda i: (i, 0)),
            pl.BlockSpec((_NROW, H), lambda i: (i, 0)),
            pl.BlockSpec((_NROW, H), lambda i: (i, 0)),
            pl.BlockSpec((H, H), lambda i: (0, 0)),
            pl.BlockSpec((1, H), lambda i: (0, 0)),
            pl.BlockSpec((H, H), lambda i: (0, 0)),
            pl.BlockSpec((1, H), lambda i: (0, 0)),
        ],
        out_specs=[
            pl.BlockSpec((_NROW, H), lambda i: (i, 0)),
            pl.BlockSpec((2, H), lambda i: (0, 0)),
        ],
        out_shape=[
            jax.ShapeDtypeStruct((N, H), F32),
            jax.ShapeDtypeStruct((2, H), F32),
        ],
        scratch_shapes=[
            pltpu.VMEM((1, H), F32),
            pltpu.VMEM((1, H), F32),
        ],
    )(h, p0, p1, w1, b1.reshape(1, H), w2, b2.reshape(1, H))


def _bn_relu_kernel(t_ref, stats_ref, g_ref, b_ref, o_ref):
    mu = stats_ref[0:1, :] * (1.0 / N)
    var = stats_ref[1:2, :] * (1.0 / N) - mu * mu
    inv = lax.rsqrt(var + 1e-5)
    o_ref[...] = jnp.maximum(
        g_ref[...] * (t_ref[...] - mu) * inv + b_ref[...], 0.0)


def _bn_relu(t, stats, g, b):
    return pl.pallas_call(
        _bn_relu_kernel,
        grid=(_NT,),
        in_specs=[
            pl.BlockSpec((_NROW, H), lambda i: (i, 0)),
            pl.BlockSpec((2, H), lambda i: (0, 0)),
            pl.BlockSpec((1, H), lambda i: (0, 0)),
            pl.BlockSpec((1, H), lambda i: (0, 0)),
        ],
        out_specs=pl.BlockSpec((_NROW, H), lambda i: (i, 0)),
        out_shape=jax.ShapeDtypeStruct((N, H), F32),
    )(t, stats, g.reshape(1, H), b.reshape(1, H))


def _final_kernel(h_ref, batch_ref, ext_ref, wx1_ref, bx1_ref, wx2_ref,
                  bx2_ref, wf1_ref, bf1_ref, wf2_ref, bf2_ref,
                  o_ref, sums, cnts):
    i = pl.program_id(0)

    @pl.when(i == 0)
    def _():
        sums[...] = jnp.zeros_like(sums)
        cnts[...] = jnp.zeros_like(cnts)

    b = batch_ref[0]
    bb = jnp.broadcast_to(b, (G, _NROW))
    gi = lax.broadcasted_iota(jnp.int32, (G, _NROW), 0)
    oh = (bb == gi).astype(F32)
    sums[...] += jnp.dot(oh, h_ref[...], preferred_element_type=F32)
    cnts[...] += jnp.dot(oh, jnp.ones((_NROW, H), F32),
                         preferred_element_type=F32)

    @pl.when(i == _NT - 1)
    def _():
        emb = sums[...] / jnp.maximum(cnts[...], 1.0)
        ext = jnp.maximum(jnp.dot(ext_ref[...], wx1_ref[...],
                                  preferred_element_type=F32)
                          + bx1_ref[...], 0.0)
        ext = jnp.dot(ext, wx2_ref[...],
                      preferred_element_type=F32) + bx2_ref[...]
        comb = jnp.concatenate([emb, ext], axis=1)
        r = jnp.maximum(jnp.dot(comb, wf1_ref[...],
                                preferred_element_type=F32)
                        + bf1_ref[...], 0.0)
        o_ref[...] = jnp.dot(r, wf2_ref[...],
                             preferred_element_type=F32) + bf2_ref[...]


def _final(h, batch, ext, wx1, bx1, wx2, bx2, wf1, bf1, wf2, bf2):
    d = ext.shape[1]
    batch3 = batch.reshape(_NT, 1, _NROW)
    return pl.pallas_call(
        _final_kernel,
        grid=(_NT,),
        in_specs=[
            pl.BlockSpec((_NROW, H), lambda i: (i, 0)),
            pl.BlockSpec((1, 1, _NROW), lambda i: (i, 0, 0)),
            pl.BlockSpec((G, d), lambda i: (0, 0)),
            pl.BlockSpec((d, H), lambda i: (0, 0)),
            pl.BlockSpec((1, H), lambda i: (0, 0)),
            pl.BlockSpec((H, H), lambda i: (0, 0)),
            pl.BlockSpec((1, H), lambda i: (0, 0)),
            pl.BlockSpec((2 * H, H), lambda i: (0, 0)),
            pl.BlockSpec((1, H), lambda i: (0, 0)),
            pl.BlockSpec((H, 1), lambda i: (0, 0)),
            pl.BlockSpec((1, 1), lambda i: (0, 0)),
        ],
        out_specs=pl.BlockSpec((G, 1), lambda i: (0, 0)),
        out_shape=jax.ShapeDtypeStruct((G, 1), F32),
        scratch_shapes=[
            pltpu.VMEM((G, H), F32),
            pltpu.VMEM((G, H), F32),
        ],
    )(h, batch3, ext, wx1, bx1.reshape(1, H), wx2, bx2.reshape(1, H),
      wf1, bf1.reshape(1, H), wf2, bf2.reshape(1, 1))


def _edge_stage(h, e, src, dst):
    mesh = plsc.VectorSubcoreMesh(core_axis_name="c", subcore_axis_name="s")
    cp = pltpu.CompilerParams()
    if "needs_layout_passes" in pltpu.CompilerParams.__dataclass_fields__:
        cp = dataclasses.replace(cp, needs_layout_passes=False)

    @functools.partial(
        pl.kernel,
        out_type=jax.ShapeDtypeStruct((2, N, H), F32),
        mesh=mesh,
        compiler_params=cp,
        scratch_types=[
            pltpu.VMEM((_CHUNK,), jnp.int32),
            pltpu.VMEM((_CHUNK,), jnp.int32),
            pltpu.VMEM((_CHUNK,), jnp.int32),
            pltpu.VMEM((_CHUNK,), jnp.int32),
            pltpu.VMEM((_CHUNK, H), F32),
            pltpu.VMEM((_CHUNK, H), F32),
            pltpu.VMEM((_CHUNK // 2, H), jnp.int32),
            pltpu.VMEM((_CHUNK // 2, H), jnp.int32),
            pltpu.VMEM_SHARED((N, H), F32),
            pltpu.SemaphoreType.DMA,
            pltpu.SemaphoreType.DMA,
            pltpu.SemaphoreType.DMA,
            pltpu.SemaphoreType.DMA,
            pltpu.SemaphoreType.DMA,
            pltpu.SemaphoreType.DMA,
            pltpu.SemaphoreType.DMA,
            pltpu.SemaphoreType.DMA,
        ],
    )
    def k(h_hbm, e_hbm, src_hbm, dst_hbm, out_hbm,
          sc0, sc1, dc0, dc1, r0, r1, e0, e1,
          acc, g0, g1, s0, s1, xs0, xs1, xd0, xd1):
        cid = lax.axis_index("c")
        sid = lax.axis_index("s")
        w = cid * 16 + sid
        srcb = (sc0, sc1)
        dstb = (dc0, dc1)
        rows = (r0, r1)
        ebuf = (e0, e1)
        gsem = (g0, g1)
        esem = (s0, s1)
        xssem = (xs0, xs1)
        xdsem = (xd0, xd1)
        base0 = w * _CPT * _CHUNK
        ebase0 = w * _CPT * (_CHUNK // 2)

        def issue_src(i, q):
            pltpu.async_copy(src_hbm.at[pl.ds(base0 + i * _CHUNK, _CHUNK)],
                             srcb[q], xssem[q])

        def wait_src(i, q):
            pltpu.make_async_copy(src_hbm.at[pl.ds(base0 + i * _CHUNK,
                                                   _CHUNK)],
                                  srcb[q], xssem[q]).wait()

        def issue_dst(i, q):
            pltpu.async_copy(dst_hbm.at[pl.ds(base0 + i * _CHUNK, _CHUNK)],
                             dstb[q], xdsem[q])

        def wait_dst(i, q):
            pltpu.make_async_copy(dst_hbm.at[pl.ds(base0 + i * _CHUNK,
                                                   _CHUNK)],
                                  dstb[q], xdsem[q]).wait()

        def issue_data(i, b):
            pltpu.async_copy(h_hbm.at[srcb[b]], rows[b], gsem[b])
            pltpu.async_copy(
                e_hbm.at[pl.ds(ebase0 + i * (_CHUNK // 2), _CHUNK // 2)],
                ebuf[b], esem[b])

        def wait_data(i, b):
            pltpu.make_async_copy(h_hbm.at[srcb[b]], rows[b], gsem[b]).wait()
            pltpu.make_async_copy(
                e_hbm.at[pl.ds(ebase0 + i * (_CHUNK // 2), _CHUNK // 2)],
                ebuf[b], esem[b]).wait()

        def compute(rb, eb):
            hmask = jnp.full((16,), -65536, jnp.int32)

            @pl.loop(0, _CHUNK // 2)
            def _(rr):
                for half in range(2):
                    r = 2 * rr + half
                    for g in range(H // 32):
                        wv = eb[rr, pl.ds(64 * half + 16 * g, 16)]
                        lo = plsc.bitcast(wv << 16, F32)
                        hi = plsc.bitcast(wv & hmask, F32)
                        sl0 = pl.ds(32 * g, 16)
                        sl1 = pl.ds(32 * g + 16, 16)
                        rb[r, sl0] = jnp.maximum(rb[r, sl0] + lo, 0.0)
                        rb[r, sl1] = jnp.maximum(rb[r, sl1] + hi, 0.0)

        issue_src(0, 0)
        issue_src(1, 1)
        issue_dst(0, 0)
        issue_dst(1, 1)
        wait_src(0, 0)
        issue_data(0, 0)

        @pl.loop(0, _CHUNK)
        def _(r):
            for j in range(H // 16):
                r1[r, pl.ds(j * 16, 16)] = jnp.zeros((16,), F32)
        row0 = sid * _RPT
        for t in range(_RPT // _CHUNK):
            pltpu.sync_copy(r1, acc.at[pl.ds(row0 + t * _CHUNK, _CHUNK)])
        rem = _RPT % _CHUNK
        if rem:
            pltpu.sync_copy(r1.at[pl.ds(0, rem)],
                            acc.at[pl.ds(row0 + _RPT - rem, rem)])

        @pl.when(sid == 15)
        def _():
            pltpu.sync_copy(r1.at[pl.ds(0, N - 16 * _RPT)],
                            acc.at[pl.ds(16 * _RPT, N - 16 * _RPT)])
        plsc.subcore_barrier()

        def step(i, b):
            @pl.when(i + 1 < _CPT)
            def _():
                wait_src(i + 1, 1 - b)
                issue_data(i + 1, 1 - b)
            wait_data(i, b)

            @pl.when(i + 2 < _CPT)
            def _():
                issue_src(i + 2, b)
            compute(rows[b], ebuf[b])
            wait_dst(i, b)
            pltpu.sync_copy(rows[b], acc.at[dstb[b]], add=True)

            @pl.when(i + 2 < _CPT)
            def _():
                issue_dst(i + 2, b)

        @pl.loop(0, _CPT, step=2)
        def _(i):
            for j in range(2):
                step(i + j, j)

        @pl.when(w < 4)
        def _():
            eb0 = (32 * _CPT + w) * _CHUNK
            pltpu.sync_copy(src_hbm.at[pl.ds(eb0, _CHUNK)], sc0)
            pltpu.sync_copy(dst_hbm.at[pl.ds(eb0, _CHUNK)], dc0)
            pltpu.async_copy(h_hbm.at[sc0], r0, g0).wait()
            pltpu.sync_copy(
                e_hbm.at[pl.ds((32 * _CPT + w) * (_CHUNK // 2), _CHUNK // 2)],
                e0)
            compute(r0, e0)
            pltpu.sync_copy(r0, acc.at[dc0], add=True)

        plsc.subcore_barrier()
        pltpu.sync_copy(acc.at[pl.ds(row0, _RPT)],
                        out_hbm.at[cid].at[pl.ds(row0, _RPT)])

        @pl.when(sid == 15)
        def _():
            pltpu.sync_copy(acc.at[pl.ds(16 * _RPT, N - 16 * _RPT)],
                            out_hbm.at[cid].at[pl.ds(16 * _RPT, N - 16 * _RPT)])

    return k(h, e, src, dst)


def kernel(x, edge_index, edge_attr, batch, externals, W_node, b_node,
           We1, be1, We2, be2, Wc1, bc1, Wc2, bc2, gamma, beta,
           Wx1, bx1, Wx2, bx2, Wf1, bf1, Wf2, bf2):
    src = edge_index[0]
    dst = edge_index[1]
    order = jnp.array([32 * g + 16 * h_ + k
                       for g in range(H // 32)
                       for k in range(16)
                       for h_ in range(2)], jnp.int32)
    h = _node_embed(x, W_node, b_node)
    e = _edge_mlp(edge_attr, We1, be1, We2[:, order], be2[order])
    e_w = lax.bitcast_convert_type(e.reshape(E, H // 2, 2),
                                   jnp.int32).reshape(E // 2, H)
    for l in range(Wc1.shape[0]):
        parts = _edge_stage(h, e_w, src, dst)
        t, stats = _node_layer(h, parts[0], parts[1],
                               Wc1[l], bc1[l], Wc2[l], bc2[l])
        h = _bn_relu(t, stats, gamma[l], beta[l])
    out = _final(h, batch, externals,
                 Wx1, bx1, Wx2, bx2, Wf1, bf1, Wf2, bf2)
    return out[:, 0]


# speedup vs baseline: 2.4041x; 2.4041x over previous
import functools

import jax
import jax.numpy as jnp
from jax import lax
from jax.experimental import pallas as pl
from jax.experimental.pallas import tpu as pltpu
from jax.experimental.pallas import tpu_sc as plsc

N = 10000
E = 320000
G = 256
H = 128
F32 = jnp.float32

_NT = 5
_NROW = N // _NT
_EROW = 2560

_CHUNK = 64
_CPT = 156
_RPT = 624


def _mm_bias_kernel(x_ref, w_ref, b_ref, o_ref):
    o_ref[...] = jnp.dot(x_ref[...], w_ref[...],
                         preferred_element_type=F32) + b_ref[...]


def _node_embed(x, w, b):
    return pl.pallas_call(
        _mm_bias_kernel,
        grid=(_NT,),
        in_specs=[
            pl.BlockSpec((_NROW, H), lambda i: (i, 0)),
            pl.BlockSpec((H, H), lambda i: (0, 0)),
            pl.BlockSpec((1, H), lambda i: (0, 0)),
        ],
        out_specs=pl.BlockSpec((_NROW, H), lambda i: (i, 0)),
        out_shape=jax.ShapeDtypeStruct((N, H), F32),
    )(x, w, b.reshape(1, H))


def _edge_mlp_kernel(a_ref, w1_ref, b1_ref, w2_ref, b2_ref, o_ref):
    t = jnp.maximum(jnp.dot(a_ref[...], w1_ref[...],
                            preferred_element_type=F32) + b1_ref[...], 0.0)
    o_ref[...] = jnp.dot(t, w2_ref[...],
                         preferred_element_type=F32) + b2_ref[...]


def _edge_mlp(a, w1, b1, w2, b2):
    d = a.shape[1]
    ne = a.shape[0]
    return pl.pallas_call(
        _edge_mlp_kernel,
        grid=(ne // _EROW,),
        in_specs=[
            pl.BlockSpec((_EROW, d), lambda i: (i, 0)),
            pl.BlockSpec((d, H), lambda i: (0, 0)),
            pl.BlockSpec((1, H), lambda i: (0, 0)),
            pl.BlockSpec((H, H), lambda i: (0, 0)),
            pl.BlockSpec((1, H), lambda i: (0, 0)),
        ],
        out_specs=pl.BlockSpec((_EROW, H), lambda i: (i, 0)),
        out_shape=jax.ShapeDtypeStruct((ne, H), F32),
    )(a, w1, b1.reshape(1, H), w2, b2.reshape(1, H))


def _make_node_layer_kernel(first):
    def body(t_in_ref, ab_ref, p0_ref, p1_ref, w1_ref, b1_ref, w2_ref, b2_ref,
             g_ref, be_ref, t_ref, abo_ref, ssum, ssq):
        i = pl.program_id(0)

        @pl.when(i == 0)
        def _():
            ssum[...] = jnp.zeros_like(ssum)
            ssq[...] = jnp.zeros_like(ssq)

        if first:
            hh = t_in_ref[...]
        else:
            hh = jnp.maximum(
                ab_ref[0:1, :] * t_in_ref[...] + ab_ref[1:2, :], 0.0)
        z = hh + p0_ref[...] + p1_ref[...]
        t = jnp.maximum(jnp.dot(z, w1_ref[...],
                                preferred_element_type=F32) + b1_ref[...], 0.0)
        t = jnp.dot(t, w2_ref[...], preferred_element_type=F32) + b2_ref[...]
        t_ref[...] = t
        ssum[...] += jnp.sum(t, axis=0, keepdims=True)
        ssq[...] += jnp.sum(t * t, axis=0, keepdims=True)

        @pl.when(i == _NT - 1)
        def _():
            mu = ssum[...] * (1.0 / N)
            var = ssq[...] * (1.0 / N) - mu * mu
            a = g_ref[...] * lax.rsqrt(var + 1e-5)
            abo_ref[0:1, :] = a
            abo_ref[1:2, :] = be_ref[...] - mu * a

    return body


def _node_layer(t_in, ab, p0, p1, w1, b1, w2, b2, g, be, first):
    return pl.pallas_call(
        _make_node_layer_kernel(first),
        grid=(_NT,),
        in_specs=[
            pl.BlockSpec((_NROW, H), lambda i: (i, 0)),
            pl.BlockSpec((2, H), lambda i: (0, 0)),
            pl.BlockSpec((_NROW, H), lambda i: (i, 0)),
            pl.BlockSpec((_NROW, H), lambda i: (i, 0)),
            pl.BlockSpec((H, H), lambda i: (0, 0)),
            pl.BlockSpec((1, H), lambda i: (0, 0)),
            pl.BlockSpec((H, H), lambda i: (0, 0)),
            pl.BlockSpec((1, H), lambda i: (0, 0)),
            pl.BlockSpec((1, H), lambda i: (0, 0)),
            pl.BlockSpec((1, H), lambda i: (0, 0)),
        ],
        out_specs=[
            pl.BlockSpec((_NROW, H), lambda i: (i, 0)),
            pl.BlockSpec((2, H), lambda i: (0, 0)),
        ],
        out_shape=[
            jax.ShapeDtypeStruct((N, H), F32),
            jax.ShapeDtypeStruct((2, H), F32),
        ],
        scratch_shapes=[
            pltpu.VMEM((1, H), F32),
            pltpu.VMEM((1, H), F32),
        ],
    )(t_in, ab, p0, p1, w1, b1.reshape(1, H), w2, b2.reshape(1, H),
      g.reshape(1, H), be.reshape(1, H))


def _final_kernel(h_ref, ab_ref, batch_ref, ext_ref, wx1_ref, bx1_ref, wx2_ref,
                  bx2_ref, wf1_ref, bf1_ref, wf2_ref, bf2_ref,
                  o_ref, sums, cnts):
    i = pl.program_id(0)

    @pl.when(i == 0)
    def _():
        sums[...] = jnp.zeros_like(sums)
        cnts[...] = jnp.zeros_like(cnts)

    hh = jnp.maximum(ab_ref[0:1, :] * h_ref[...] + ab_ref[1:2, :], 0.0)
    b = batch_ref[0]
    bb = jnp.broadcast_to(b, (G, _NROW))
    gi = lax.broadcasted_iota(jnp.int32, (G, _NROW), 0)
    oh = (bb == gi).astype(F32)
    sums[...] += jnp.dot(oh, hh, preferred_element_type=F32)
    cnts[...] += jnp.dot(oh, jnp.ones((_NROW, H), F32),
                         preferred_element_type=F32)

    @pl.when(i == _NT - 1)
    def _():
        emb = sums[...] / jnp.maximum(cnts[...], 1.0)
        ext = jnp.maximum(jnp.dot(ext_ref[...], wx1_ref[...],
                                  preferred_element_type=F32)
                          + bx1_ref[...], 0.0)
        ext = jnp.dot(ext, wx2_ref[...],
                      preferred_element_type=F32) + bx2_ref[...]
        comb = jnp.concatenate([emb, ext], axis=1)
        r = jnp.maximum(jnp.dot(comb, wf1_ref[...],
                                preferred_element_type=F32)
                        + bf1_ref[...], 0.0)
        o_ref[...] = jnp.dot(r, wf2_ref[...],
                             preferred_element_type=F32) + bf2_ref[...]


def _final(h, ab, batch, ext, wx1, bx1, wx2, bx2, wf1, bf1, wf2, bf2):
    d = ext.shape[1]
    batch3 = batch.reshape(_NT, 1, _NROW)
    return pl.pallas_call(
        _final_kernel,
        grid=(_NT,),
        in_specs=[
            pl.BlockSpec((_NROW, H), lambda i: (i, 0)),
            pl.BlockSpec((2, H), lambda i: (0, 0)),
            pl.BlockSpec((1, 1, _NROW), lambda i: (i, 0, 0)),
            pl.BlockSpec((G, d), lambda i: (0, 0)),
            pl.BlockSpec((d, H), lambda i: (0, 0)),
            pl.BlockSpec((1, H), lambda i: (0, 0)),
            pl.BlockSpec((H, H), lambda i: (0, 0)),
            pl.BlockSpec((1, H), lambda i: (0, 0)),
            pl.BlockSpec((2 * H, H), lambda i: (0, 0)),
            pl.BlockSpec((1, H), lambda i: (0, 0)),
            pl.BlockSpec((H, 1), lambda i: (0, 0)),
            pl.BlockSpec((1, 1), lambda i: (0, 0)),
        ],
        out_specs=pl.BlockSpec((G, 1), lambda i: (0, 0)),
        out_shape=jax.ShapeDtypeStruct((G, 1), F32),
        scratch_shapes=[
            pltpu.VMEM((G, H), F32),
            pltpu.VMEM((G, H), F32),
        ],
    )(h, ab, batch3, ext, wx1, bx1.reshape(1, H), wx2, bx2.reshape(1, H),
      wf1, bf1.reshape(1, H), wf2, bf2.reshape(1, 1))


def _edge_stage(t_in, e, src, dst, ab, inner_relu):
    mesh = plsc.VectorSubcoreMesh(core_axis_name="c", subcore_axis_name="s")

    @functools.partial(
        pl.kernel,
        out_type=jax.ShapeDtypeStruct((2, N, H), F32),
        mesh=mesh,
        scratch_types=[
            pltpu.VMEM((_CHUNK,), jnp.int32),
            pltpu.VMEM((_CHUNK,), jnp.int32),
            pltpu.VMEM((_CHUNK,), jnp.int32),
            pltpu.VMEM((_CHUNK,), jnp.int32),
            pltpu.VMEM((_CHUNK, H), F32),
            pltpu.VMEM((_CHUNK, H), F32),
            pltpu.VMEM((_CHUNK, H), F32),
            pltpu.VMEM((_CHUNK, H), F32),
            pltpu.VMEM((2, H), F32),
            pltpu.VMEM_SHARED((N, H), F32),
            pltpu.SemaphoreType.DMA,
            pltpu.SemaphoreType.DMA,
            pltpu.SemaphoreType.DMA,
            pltpu.SemaphoreType.DMA,
            pltpu.SemaphoreType.DMA,
            pltpu.SemaphoreType.DMA,
            pltpu.SemaphoreType.DMA,
            pltpu.SemaphoreType.DMA,
        ],
    )
    def k(t_hbm, e_hbm, src_hbm, dst_hbm, ab_hbm, out_hbm,
          sc0, sc1, dc0, dc1, r0, r1, e0, e1, abv,
          acc, g0, g1, s0, s1, xs0, xs1, xd0, xd1):
        cid = lax.axis_index("c")
        sid = lax.axis_index("s")
        w = cid * 16 + sid
        srcb = (sc0, sc1)
        dstb = (dc0, dc1)
        rows = (r0, r1)
        ebuf = (e0, e1)
        gsem = (g0, g1)
        esem = (s0, s1)
        xssem = (xs0, xs1)
        xdsem = (xd0, xd1)
        base0 = w * _CPT * _CHUNK

        def issue_src(i, q):
            pltpu.async_copy(src_hbm.at[pl.ds(base0 + i * _CHUNK, _CHUNK)],
                             srcb[q], xssem[q])

        def wait_src(i, q):
            pltpu.make_async_copy(src_hbm.at[pl.ds(base0 + i * _CHUNK,
                                                   _CHUNK)],
                                  srcb[q], xssem[q]).wait()

        def issue_dst(i, q):
            pltpu.async_copy(dst_hbm.at[pl.ds(base0 + i * _CHUNK, _CHUNK)],
                             dstb[q], xdsem[q])

        def wait_dst(i, q):
            pltpu.make_async_copy(dst_hbm.at[pl.ds(base0 + i * _CHUNK,
                                                   _CHUNK)],
                                  dstb[q], xdsem[q]).wait()

        def issue_data(i, b):
            pltpu.async_copy(t_hbm.at[srcb[b]], rows[b], gsem[b])
            pltpu.async_copy(e_hbm.at[pl.ds(base0 + i * _CHUNK, _CHUNK)],
                             ebuf[b], esem[b])

        def wait_data(i, b):
            pltpu.make_async_copy(t_hbm.at[srcb[b]], rows[b], gsem[b]).wait()
            pltpu.make_async_copy(e_hbm.at[pl.ds(base0 + i * _CHUNK, _CHUNK)],
                                  ebuf[b], esem[b]).wait()

        issue_src(0, 0)
        issue_src(1, 1)
        issue_dst(0, 0)
        issue_dst(1, 1)
        pltpu.sync_copy(ab_hbm, abv)
        wait_src(0, 0)
        issue_data(0, 0)

        av = [abv[0, pl.ds(16 * j, 16)] for j in range(H // 16)]
        bv = [abv[1, pl.ds(16 * j, 16)] for j in range(H // 16)]

        def compute(rb, eb):
            @pl.loop(0, _CHUNK)
            def _(r):
                for j in range(H // 16):
                    sl = pl.ds(16 * j, 16)
                    v = rb[r, sl] * av[j] + bv[j]
                    if inner_relu:
                        v = jnp.maximum(v, 0.0)
                    rb[r, sl] = jnp.maximum(v + eb[r, sl], 0.0)

        @pl.loop(0, _CHUNK)
        def _(r):
            for j in range(H // 16):
                r1[r, pl.ds(j * 16, 16)] = jnp.zeros((16,), F32)
        row0 = sid * _RPT
        for t in range(_RPT // _CHUNK):
            pltpu.sync_copy(r1, acc.at[pl.ds(row0 + t * _CHUNK, _CHUNK)])
        rem = _RPT % _CHUNK
        if rem:
            pltpu.sync_copy(r1.at[pl.ds(0, rem)],
                            acc.at[pl.ds(row0 + _RPT - rem, rem)])

        @pl.when(sid == 15)
        def _():
            pltpu.sync_copy(r1.at[pl.ds(0, N - 16 * _RPT)],
                            acc.at[pl.ds(16 * _RPT, N - 16 * _RPT)])
        plsc.subcore_barrier()

        def step(i, b):
            @pl.when(i + 1 < _CPT)
            def _():
                wait_src(i + 1, 1 - b)
                issue_data(i + 1, 1 - b)
            wait_data(i, b)

            @pl.when(i + 2 < _CPT)
            def _():
                issue_src(i + 2, b)
            compute(rows[b], ebuf[b])
            wait_dst(i, b)
            pltpu.sync_copy(rows[b], acc.at[dstb[b]], add=True)

            @pl.when(i + 2 < _CPT)
            def _():
                issue_dst(i + 2, b)

        @pl.loop(0, _CPT, step=2)
        def _(i):
            for j in range(2):
                step(i + j, j)

        @pl.when(w < 8)
        def _():
            eb0 = (32 * _CPT + w) * _CHUNK
            pltpu.sync_copy(src_hbm.at[pl.ds(eb0, _CHUNK)], sc0)
            pltpu.sync_copy(dst_hbm.at[pl.ds(eb0, _CHUNK)], dc0)
            pltpu.async_copy(t_hbm.at[sc0], r0, g0).wait()
            pltpu.sync_copy(e_hbm.at[pl.ds(eb0, _CHUNK)], e0)
            compute(r0, e0)
            pltpu.sync_copy(r0, acc.at[dc0], add=True)

        plsc.subcore_barrier()
        pltpu.sync_copy(acc.at[pl.ds(row0, _RPT)],
                        out_hbm.at[cid].at[pl.ds(row0, _RPT)])

        @pl.when(sid == 15)
        def _():
            pltpu.sync_copy(acc.at[pl.ds(16 * _RPT, N - 16 * _RPT)],
                            out_hbm.at[cid].at[pl.ds(16 * _RPT, N - 16 * _RPT)])

    return k(t_in, e, src, dst, ab)


def kernel(x, edge_index, edge_attr, batch, externals, W_node, b_node,
           We1, be1, We2, be2, Wc1, bc1, Wc2, bc2, gamma, beta,
           Wx1, bx1, Wx2, bx2, Wf1, bf1, Wf2, bf2):
    src = edge_index[0]
    dst = edge_index[1]
    t = _node_embed(x, W_node, b_node)
    e = _edge_mlp(edge_attr, We1, be1, We2, be2)
    ab = jnp.concatenate([jnp.ones((1, H), F32), jnp.zeros((1, H), F32)])
    for l in range(Wc1.shape[0]):
        first = l == 0
        parts = _edge_stage(t, e, src, dst, ab, inner_relu=not first)
        t, ab = _node_layer(t, ab, parts[0], parts[1],
                            Wc1[l], bc1[l], Wc2[l], bc2[l],
                            gamma[l], beta[l], first)
    out = _final(t, ab, batch, externals,
                 Wx1, bx1, Wx2, bx2, Wf1, bf1, Wf2, bf2)
    return out[:, 0]


# speedup vs baseline: 2.7577x; 1.1471x over previous
import functools

import jax
import jax.numpy as jnp
from jax import lax
from jax.experimental import pallas as pl
from jax.experimental.pallas import tpu as pltpu
from jax.experimental.pallas import tpu_sc as plsc

N = 10000
E = 320000
G = 256
H = 128
F32 = jnp.float32

_NT = 5
_NROW = N // _NT
_EROW = 2560

_CHUNK = 64
_CPT = 156
_RPT = 624


def _mm_bias_kernel(x_ref, w_ref, b_ref, o_ref):
    o_ref[...] = jnp.dot(x_ref[...], w_ref[...],
                         preferred_element_type=F32) + b_ref[...]


def _node_embed(x, w, b):
    return pl.pallas_call(
        _mm_bias_kernel,
        grid=(_NT,),
        in_specs=[
            pl.BlockSpec((_NROW, H), lambda i: (i, 0)),
            pl.BlockSpec((H, H), lambda i: (0, 0)),
            pl.BlockSpec((1, H), lambda i: (0, 0)),
        ],
        out_specs=pl.BlockSpec((_NROW, H), lambda i: (i, 0)),
        out_shape=jax.ShapeDtypeStruct((N, H), F32),
    )(x, w, b.reshape(1, H))


def _edge_mlp_kernel(at_ref, w1_ref, b1_ref, w2_ref, b2_ref, o_ref):
    t = jnp.maximum(
        lax.dot_general(at_ref[...], w1_ref[...], (((0,), (0,)), ((), ())),
                        preferred_element_type=F32) + b1_ref[...], 0.0)
    o_ref[...] = jnp.dot(t, w2_ref[...],
                         preferred_element_type=F32) + b2_ref[...]


def _edge_mlp(a, w1, b1, w2, b2):
    d = a.shape[1]
    ne = a.shape[0]
    return pl.pallas_call(
        _edge_mlp_kernel,
        grid=(ne // _EROW,),
        in_specs=[
            pl.BlockSpec((d, _EROW), lambda i: (0, i)),
            pl.BlockSpec((d, H), lambda i: (0, 0)),
            pl.BlockSpec((1, H), lambda i: (0, 0)),
            pl.BlockSpec((H, H), lambda i: (0, 0)),
            pl.BlockSpec((1, H), lambda i: (0, 0)),
        ],
        out_specs=pl.BlockSpec((_EROW, H), lambda i: (i, 0)),
        out_shape=jax.ShapeDtypeStruct((ne, H), F32),
    )(a.T, w1, b1.reshape(1, H), w2, b2.reshape(1, H))


def _make_node_layer_kernel(first):
    def body(t_in_ref, ab_ref, p0_ref, p1_ref, w1_ref, b1_ref, w2_ref, b2_ref,
             g_ref, be_ref, t_ref, abo_ref, ssum, ssq):
        i = pl.program_id(0)

        @pl.when(i == 0)
        def _():
            ssum[...] = jnp.zeros_like(ssum)
            ssq[...] = jnp.zeros_like(ssq)

        if first:
            hh = t_in_ref[...]
        else:
            hh = jnp.maximum(
                ab_ref[0:1, :] * t_in_ref[...] + ab_ref[1:2, :], 0.0)
        z = hh + p0_ref[...] + p1_ref[...]
        t = jnp.maximum(jnp.dot(z, w1_ref[...],
                                preferred_element_type=F32) + b1_ref[...], 0.0)
        t = jnp.dot(t, w2_ref[...], preferred_element_type=F32) + b2_ref[...]
        t_ref[...] = t
        ssum[...] += jnp.sum(t, axis=0, keepdims=True)
        ssq[...] += jnp.sum(t * t, axis=0, keepdims=True)

        @pl.when(i == _NT - 1)
        def _():
            mu = ssum[...] * (1.0 / N)
            var = ssq[...] * (1.0 / N) - mu * mu
            a = g_ref[...] * lax.rsqrt(var + 1e-5)
            abo_ref[0:1, :] = a
            abo_ref[1:2, :] = be_ref[...] - mu * a

    return body


def _node_layer(t_in, ab, p0, p1, w1, b1, w2, b2, g, be, first):
    return pl.pallas_call(
        _make_node_layer_kernel(first),
        grid=(_NT,),
        in_specs=[
            pl.BlockSpec((_NROW, H), lambda i: (i, 0)),
            pl.BlockSpec((2, H), lambda i: (0, 0)),
            pl.BlockSpec((_NROW, H), lambda i: (i, 0)),
            pl.BlockSpec((_NROW, H), lambda i: (i, 0)),
            pl.BlockSpec((H, H), lambda i: (0, 0)),
            pl.BlockSpec((1, H), lambda i: (0, 0)),
            pl.BlockSpec((H, H), lambda i: (0, 0)),
            pl.BlockSpec((1, H), lambda i: (0, 0)),
            pl.BlockSpec((1, H), lambda i: (0, 0)),
            pl.BlockSpec((1, H), lambda i: (0, 0)),
        ],
        out_specs=[
            pl.BlockSpec((_NROW, H), lambda i: (i, 0)),
            pl.BlockSpec((2, H), lambda i: (0, 0)),
        ],
        out_shape=[
            jax.ShapeDtypeStruct((N, H), F32),
            jax.ShapeDtypeStruct((2, H), F32),
        ],
        scratch_shapes=[
            pltpu.VMEM((1, H), F32),
            pltpu.VMEM((1, H), F32),
        ],
    )(t_in, ab, p0, p1, w1, b1.reshape(1, H), w2, b2.reshape(1, H),
      g.reshape(1, H), be.reshape(1, H))


def _final_kernel(h_ref, ab_ref, batch_ref, ext_ref, wx1_ref, bx1_ref, wx2_ref,
                  bx2_ref, wf1_ref, bf1_ref, wf2_ref, bf2_ref,
                  o_ref, sums, cnts):
    i = pl.program_id(0)

    @pl.when(i == 0)
    def _():
        sums[...] = jnp.zeros_like(sums)
        cnts[...] = jnp.zeros_like(cnts)

    hh = jnp.maximum(ab_ref[0:1, :] * h_ref[...] + ab_ref[1:2, :], 0.0)
    b = batch_ref[0]
    bb = jnp.broadcast_to(b, (G, _NROW))
    gi = lax.broadcasted_iota(jnp.int32, (G, _NROW), 0)
    oh = (bb == gi).astype(F32)
    sums[...] += jnp.dot(oh, hh, preferred_element_type=F32)
    cnts[...] += jnp.dot(oh, jnp.ones((_NROW, H), F32),
                         preferred_element_type=F32)

    @pl.when(i == _NT - 1)
    def _():
        emb = sums[...] / jnp.maximum(cnts[...], 1.0)
        ext = jnp.maximum(jnp.dot(ext_ref[...], wx1_ref[...],
                                  preferred_element_type=F32)
                          + bx1_ref[...], 0.0)
        ext = jnp.dot(ext, wx2_ref[...],
                      preferred_element_type=F32) + bx2_ref[...]
        comb = jnp.concatenate([emb, ext], axis=1)
        r = jnp.maximum(jnp.dot(comb, wf1_ref[...],
                                preferred_element_type=F32)
                        + bf1_ref[...], 0.0)
        o_ref[...] = jnp.dot(r, wf2_ref[...],
                             preferred_element_type=F32) + bf2_ref[...]


def _final(h, ab, batch, ext, wx1, bx1, wx2, bx2, wf1, bf1, wf2, bf2):
    d = ext.shape[1]
    batch3 = batch.reshape(_NT, 1, _NROW)
    return pl.pallas_call(
        _final_kernel,
        grid=(_NT,),
        in_specs=[
            pl.BlockSpec((_NROW, H), lambda i: (i, 0)),
            pl.BlockSpec((2, H), lambda i: (0, 0)),
            pl.BlockSpec((1, 1, _NROW), lambda i: (i, 0, 0)),
            pl.BlockSpec((G, d), lambda i: (0, 0)),
            pl.BlockSpec((d, H), lambda i: (0, 0)),
            pl.BlockSpec((1, H), lambda i: (0, 0)),
            pl.BlockSpec((H, H), lambda i: (0, 0)),
            pl.BlockSpec((1, H), lambda i: (0, 0)),
            pl.BlockSpec((2 * H, H), lambda i: (0, 0)),
            pl.BlockSpec((1, H), lambda i: (0, 0)),
            pl.BlockSpec((H, 1), lambda i: (0, 0)),
            pl.BlockSpec((1, 1), lambda i: (0, 0)),
        ],
        out_specs=pl.BlockSpec((G, 1), lambda i: (0, 0)),
        out_shape=jax.ShapeDtypeStruct((G, 1), F32),
        scratch_shapes=[
            pltpu.VMEM((G, H), F32),
            pltpu.VMEM((G, H), F32),
        ],
    )(h, ab, batch3, ext, wx1, bx1.reshape(1, H), wx2, bx2.reshape(1, H),
      wf1, bf1.reshape(1, H), wf2, bf2.reshape(1, 1))


def _edge_stage(t_in, e, src, dst, ab, inner_relu):
    mesh = plsc.VectorSubcoreMesh(core_axis_name="c", subcore_axis_name="s")

    @functools.partial(
        pl.kernel,
        out_type=jax.ShapeDtypeStruct((2, N, H), F32),
        mesh=mesh,
        scratch_types=[
            pltpu.VMEM((_CHUNK,), jnp.int32),
            pltpu.VMEM((_CHUNK,), jnp.int32),
            pltpu.VMEM((_CHUNK,), jnp.int32),
            pltpu.VMEM((_CHUNK,), jnp.int32),
            pltpu.VMEM((_CHUNK, H), F32),
            pltpu.VMEM((_CHUNK, H), F32),
            pltpu.VMEM((_CHUNK, H), F32),
            pltpu.VMEM((_CHUNK, H), F32),
            pltpu.VMEM((2, H), F32),
            pltpu.VMEM_SHARED((N, H), F32),
            pltpu.SemaphoreType.DMA,
            pltpu.SemaphoreType.DMA,
            pltpu.SemaphoreType.DMA,
            pltpu.SemaphoreType.DMA,
            pltpu.SemaphoreType.DMA,
            pltpu.SemaphoreType.DMA,
            pltpu.SemaphoreType.DMA,
            pltpu.SemaphoreType.DMA,
        ],
    )
    def k(t_hbm, e_hbm, src_hbm, dst_hbm, ab_hbm, out_hbm,
          sc0, sc1, dc0, dc1, r0, r1, e0, e1, abv,
          acc, g0, g1, s0, s1, xs0, xs1, xd0, xd1):
        cid = lax.axis_index("c")
        sid = lax.axis_index("s")
        w = cid * 16 + sid
        srcb = (sc0, sc1)
        dstb = (dc0, dc1)
        rows = (r0, r1)
        ebuf = (e0, e1)
        gsem = (g0, g1)
        esem = (s0, s1)
        xssem = (xs0, xs1)
        xdsem = (xd0, xd1)
        base0 = w * _CPT * _CHUNK

        def issue_src(i, q):
            pltpu.async_copy(src_hbm.at[pl.ds(base0 + i * _CHUNK, _CHUNK)],
                             srcb[q], xssem[q])

        def wait_src(i, q):
            pltpu.make_async_copy(src_hbm.at[pl.ds(base0 + i * _CHUNK,
                                                   _CHUNK)],
                                  srcb[q], xssem[q]).wait()

        def issue_dst(i, q):
            pltpu.async_copy(dst_hbm.at[pl.ds(base0 + i * _CHUNK, _CHUNK)],
                             dstb[q], xdsem[q])

        def wait_dst(i, q):
            pltpu.make_async_copy(dst_hbm.at[pl.ds(base0 + i * _CHUNK,
                                                   _CHUNK)],
                                  dstb[q], xdsem[q]).wait()

        def issue_data(i, b):
            pltpu.async_copy(t_hbm.at[srcb[b]], rows[b], gsem[b])
            pltpu.async_copy(e_hbm.at[pl.ds(base0 + i * _CHUNK, _CHUNK)],
                             ebuf[b], esem[b])

        def wait_data(i, b):
            pltpu.make_async_copy(t_hbm.at[srcb[b]], rows[b], gsem[b]).wait()
            pltpu.make_async_copy(e_hbm.at[pl.ds(base0 + i * _CHUNK, _CHUNK)],
                                  ebuf[b], esem[b]).wait()

        issue_src(0, 0)
        issue_src(1, 1)
        issue_dst(0, 0)
        issue_dst(1, 1)
        pltpu.sync_copy(ab_hbm, abv)
        wait_src(0, 0)
        issue_data(0, 0)

        av = [abv[0, pl.ds(16 * j, 16)] for j in range(H // 16)]
        bv = [abv[1, pl.ds(16 * j, 16)] for j in range(H // 16)]

        def compute(rb, eb):
            @pl.loop(0, _CHUNK)
            def _(r):
                for j in range(H // 16):
                    sl = pl.ds(16 * j, 16)
                    v = rb[r, sl] * av[j] + bv[j]
                    if inner_relu:
                        v = jnp.maximum(v, 0.0)
                    rb[r, sl] = jnp.maximum(v + eb[r, sl], 0.0)

        @pl.loop(0, _CHUNK)
        def _(r):
            for j in range(H // 16):
                r1[r, pl.ds(j * 16, 16)] = jnp.zeros((16,), F32)
        row0 = sid * _RPT
        for t in range(_RPT // _CHUNK):
            pltpu.sync_copy(r1, acc.at[pl.ds(row0 + t * _CHUNK, _CHUNK)])
        rem = _RPT % _CHUNK
        if rem:
            pltpu.sync_copy(r1.at[pl.ds(0, rem)],
                            acc.at[pl.ds(row0 + _RPT - rem, rem)])

        @pl.when(sid == 15)
        def _():
            pltpu.sync_copy(r1.at[pl.ds(0, N - 16 * _RPT)],
                            acc.at[pl.ds(16 * _RPT, N - 16 * _RPT)])
        plsc.subcore_barrier()

        def step(i, b):
            @pl.when(i + 1 < _CPT)
            def _():
                wait_src(i + 1, 1 - b)
                issue_data(i + 1, 1 - b)
            wait_data(i, b)

            @pl.when(i + 2 < _CPT)
            def _():
                issue_src(i + 2, b)
            compute(rows[b], ebuf[b])
            wait_dst(i, b)
            pltpu.sync_copy(rows[b], acc.at[dstb[b]], add=True)

            @pl.when(i + 2 < _CPT)
            def _():
                issue_dst(i + 2, b)

        @pl.loop(0, _CPT, step=2)
        def _(i):
            for j in range(2):
                step(i + j, j)

        @pl.when(w < 8)
        def _():
            eb0 = (32 * _CPT + w) * _CHUNK
            pltpu.sync_copy(src_hbm.at[pl.ds(eb0, _CHUNK)], sc0)
            pltpu.sync_copy(dst_hbm.at[pl.ds(eb0, _CHUNK)], dc0)
            pltpu.async_copy(t_hbm.at[sc0], r0, g0).wait()
            pltpu.sync_copy(e_hbm.at[pl.ds(eb0, _CHUNK)], e0)
            compute(r0, e0)
            pltpu.sync_copy(r0, acc.at[dc0], add=True)

        plsc.subcore_barrier()
        pltpu.sync_copy(acc.at[pl.ds(row0, _RPT)],
                        out_hbm.at[cid].at[pl.ds(row0, _RPT)])

        @pl.when(sid == 15)
        def _():
            pltpu.sync_copy(acc.at[pl.ds(16 * _RPT, N - 16 * _RPT)],
                            out_hbm.at[cid].at[pl.ds(16 * _RPT, N - 16 * _RPT)])

    return k(t_in, e, src, dst, ab)


def kernel(x, edge_index, edge_attr, batch, externals, W_node, b_node,
           We1, be1, We2, be2, Wc1, bc1, Wc2, bc2, gamma, beta,
           Wx1, bx1, Wx2, bx2, Wf1, bf1, Wf2, bf2):
    src = edge_index[0]
    dst = edge_index[1]
    t = _node_embed(x, W_node, b_node)
    e = _edge_mlp(edge_attr, We1, be1, We2, be2)
    ab = jnp.concatenate([jnp.ones((1, H), F32), jnp.zeros((1, H), F32)])
    for l in range(Wc1.shape[0]):
        first = l == 0
        parts = _edge_stage(t, e, src, dst, ab, inner_relu=not first)
        t, ab = _node_layer(t, ab, parts[0], parts[1],
                            Wc1[l], bc1[l], Wc2[l], bc2[l],
                            gamma[l], beta[l], first)
    out = _final(t, ab, batch, externals,
                 Wx1, bx1, Wx2, bx2, Wf1, bf1, Wf2, bf2)
    return out[:, 0]
